# Initial kernel scaffold; baseline (speedup 1.0000x reference)
import jax
import jax.numpy as jnp
from jax.experimental import pallas as pl


def kernel(nf, ef, edge_index, W_ih_e, W_hh_e, b_ih_e, b_hh_e, W_ih_n, W_hh_n, b_ih_n, b_hh_n, W_attn, b_attn):
    raise NotImplementedError("write your pallas kernel here")


# speedup vs baseline: 5.6791x; 5.6791x over previous
import functools

import jax
import jax.numpy as jnp
from jax import lax
from jax.experimental import pallas as pl
from jax.experimental.pallas import tpu as pltpu
from jax.experimental.pallas import tpu_sc as plsc

N_NODES = 10000
N_EDGES = 320000
NODE_DIM = 128
EDGE_DIM = 16
N_ITERS = 3

NC = 2
NS = 16
NW = NC * NS
CH = 80
EW = N_EDGES // NW
NJ = EW // CH
N_PAD = 10240
NROW = N_PAD // NS

SD = 64
MW = 32

def _sc_gather_body(s_hbm, d_hbm, src_hbm, dst_hbm, g_hbm,
                    src_v, dst_v, srows, drows, sem_s, sem_d):
    wid = lax.axis_index("s") * NC + lax.axis_index("c")
    base = wid * EW
    pltpu.sync_copy(src_hbm.at[wid], src_v)
    pltpu.sync_copy(dst_hbm.at[wid], dst_v)

    def chunk(j, carry):
        cs = pltpu.async_copy(s_hbm.at[src_v.at[j]], srows, sem_s)
        cd = pltpu.async_copy(d_hbm.at[dst_v.at[j]], drows, sem_d)
        cs.wait()
        cd.wait()

        def add_row(i, c):
            for k in range(SD // 16):
                sl = pl.ds(k * 16, 16)
                srows[i, sl] = srows[i, sl] + drows[i, sl]
            return c

        lax.fori_loop(0, CH, add_row, 0)
        pltpu.sync_copy(srows, g_hbm.at[pl.ds(base + j * CH, CH)])
        return carry

    lax.fori_loop(0, NJ, chunk, 0)


def _sc_scatter_body(msg_hbm, dst_hbm, zero_hbm, out_hbm,
                     dst_v, msg_v, acc_sh, sem):
    c = lax.axis_index("c")
    s = lax.axis_index("s")
    wid = s * NC + c
    base = wid * EW

    pltpu.sync_copy(zero_hbm.at[pl.ds(s * NROW, NROW)],
                    acc_sh.at[pl.ds(s * NROW, NROW)])
    plsc.subcore_barrier()

    pltpu.sync_copy(dst_hbm.at[wid], dst_v)

    def chunk(j, carry):
        pltpu.sync_copy(msg_hbm.at[pl.ds(base + j * CH, CH)], msg_v)
        pltpu.sync_copy(msg_v, acc_sh.at[dst_v.at[j]], add=True)
        return carry

    lax.fori_loop(0, NJ, chunk, 0)
    plsc.subcore_barrier()

    pltpu.sync_copy(acc_sh.at[pl.ds(s * NROW, NROW)],
                    out_hbm.at[c, pl.ds(s * NROW, NROW)])


@functools.lru_cache(maxsize=None)
def _build_sc_kernels():
    mesh = plsc.VectorSubcoreMesh(core_axis_name="c", subcore_axis_name="s",
                                  num_cores=NC, num_subcores=NS)
    sc_gather = pl.kernel(
        _sc_gather_body,
        out_type=jax.ShapeDtypeStruct((N_EDGES, SD), jnp.float32),
        mesh=mesh,
        compiler_params=pltpu.CompilerParams(use_tc_tiling_on_sc=False),
        scratch_types=[
            pltpu.VMEM((NJ, CH), jnp.int32),
            pltpu.VMEM((NJ, CH), jnp.int32),
            pltpu.VMEM((CH, SD), jnp.float32),
            pltpu.VMEM((CH, SD), jnp.float32),
            pltpu.SemaphoreType.DMA,
            pltpu.SemaphoreType.DMA,
        ],
    )
    sc_scatter = pl.kernel(
        _sc_scatter_body,
        out_type=jax.ShapeDtypeStruct((NC, N_PAD, MW), jnp.float32),
        mesh=mesh,
        compiler_params=pltpu.CompilerParams(use_tc_tiling_on_sc=False),
        scratch_types=[
            pltpu.VMEM((NJ, CH), jnp.int32),
            pltpu.VMEM((CH, MW), jnp.float32),
            pltpu.VMEM_SHARED((N_PAD, MW), jnp.float32),
            pltpu.SemaphoreType.DMA,
        ],
    )
    return sc_gather, sc_scatter


def _edge_body(g_ref, ef_ref, whh_ref, ae_ref, bhh_ref, uef_ref, msg_ref):
    g = g_ref[...]
    ef = ef_ref[...]
    gh = lax.dot_general(ef, whh_ref[...], (((1,), (1,)), ((), ())),
                         preferred_element_type=jnp.float32) + bhh_ref[...]
    gi = g[:, :48]
    r = jax.nn.sigmoid(gi[:, 0:16] + gh[:, 0:16])
    z = jax.nn.sigmoid(gi[:, 16:32] + gh[:, 16:32])
    n = jnp.tanh(gi[:, 32:48] + r * gh[:, 32:48])
    uef = (1.0 - z) * n + z * ef
    logit = g[:, 48:49] + lax.dot_general(
        ef, ae_ref[...], (((1,), (1,)), ((), ())),
        preferred_element_type=jnp.float32)
    w = jnp.exp(logit)
    uef_ref[...] = uef
    be = uef.shape[0]
    msg_ref[...] = jnp.concatenate(
        [uef * w, w, jnp.zeros((be, MW - 17), jnp.float32)], axis=1)


def _node_body(a0_ref, a1_ref, nf_ref, wih_ref, whh_ref, bih_ref, bhh_ref,
               ws_ref, wd_ref, bs_ref, nfo_ref, s_ref, d_ref):
    a0 = a0_ref[...]
    a1 = a1_ref[...]
    nf = nf_ref[...]
    num = a0[:, :16] + a1[:, :16]
    den = a0[:, 16:17] + a1[:, 16:17]
    agg = jnp.where(den > 0.0, num / jnp.where(den > 0.0, den, 1.0), 0.0)
    gi = lax.dot_general(agg, wih_ref[...], (((1,), (1,)), ((), ())),
                         preferred_element_type=jnp.float32) + bih_ref[...]
    gh = lax.dot_general(nf, whh_ref[...], (((1,), (1,)), ((), ())),
                         preferred_element_type=jnp.float32) + bhh_ref[...]
    r = jax.nn.sigmoid(gi[:, 0:128] + gh[:, 0:128])
    z = jax.nn.sigmoid(gi[:, 128:256] + gh[:, 128:256])
    n = jnp.tanh(gi[:, 256:384] + r * gh[:, 256:384])
    nfo = (1.0 - z) * n + z * nf
    nfo_ref[...] = nfo
    s_ref[...] = lax.dot_general(nfo, ws_ref[...], (((1,), (1,)), ((), ())),
                                 preferred_element_type=jnp.float32) + bs_ref[...]
    d_ref[...] = lax.dot_general(nfo, wd_ref[...], (((1,), (1,)), ((), ())),
                                 preferred_element_type=jnp.float32)


def _proj_body(nf_ref, ws_ref, wd_ref, bs_ref, s_ref, d_ref):
    nf = nf_ref[...]
    s_ref[...] = lax.dot_general(nf, ws_ref[...], (((1,), (1,)), ((), ())),
                                 preferred_element_type=jnp.float32) + bs_ref[...]
    d_ref[...] = lax.dot_general(nf, wd_ref[...], (((1,), (1,)), ((), ())),
                                 preferred_element_type=jnp.float32)


_BE = 8000
_BN = 2000


def _full(shape):
    return pl.BlockSpec(shape, lambda i: (0,) * len(shape))


def _rows(shape):
    return pl.BlockSpec(shape, lambda i: (i,) + (0,) * (len(shape) - 1))


_edge_pass = pl.pallas_call(
    _edge_body,
    grid=(N_EDGES // _BE,),
    in_specs=[
        _rows((_BE, SD)),
        _rows((_BE, EDGE_DIM)),
        _full((48, EDGE_DIM)),
        _full((1, EDGE_DIM)),
        _full((1, 48)),
    ],
    out_specs=[_rows((_BE, EDGE_DIM)), _rows((_BE, MW))],
    out_shape=[
        jax.ShapeDtypeStruct((N_EDGES, EDGE_DIM), jnp.float32),
        jax.ShapeDtypeStruct((N_EDGES, MW), jnp.float32),
    ],
)

_node_pass = pl.pallas_call(
    _node_body,
    grid=(N_NODES // _BN,),
    in_specs=[
        _rows((_BN, MW)),
        _rows((_BN, MW)),
        _rows((_BN, NODE_DIM)),
        _full((3 * NODE_DIM, EDGE_DIM)),
        _full((3 * NODE_DIM, NODE_DIM)),
        _full((1, 3 * NODE_DIM)),
        _full((1, 3 * NODE_DIM)),
        _full((SD, NODE_DIM)),
        _full((SD, NODE_DIM)),
        _full((1, SD)),
    ],
    out_specs=[_rows((_BN, NODE_DIM)), _rows((_BN, SD)), _rows((_BN, SD))],
    out_shape=[
        jax.ShapeDtypeStruct((N_NODES, NODE_DIM), jnp.float32),
        jax.ShapeDtypeStruct((N_NODES, SD), jnp.float32),
        jax.ShapeDtypeStruct((N_NODES, SD), jnp.float32),
    ],
)

_proj_pass = pl.pallas_call(
    _proj_body,
    grid=(N_NODES // _BN,),
    in_specs=[
        _rows((_BN, NODE_DIM)),
        _full((SD, NODE_DIM)),
        _full((SD, NODE_DIM)),
        _full((1, SD)),
    ],
    out_specs=[_rows((_BN, SD)), _rows((_BN, SD))],
    out_shape=[
        jax.ShapeDtypeStruct((N_NODES, SD), jnp.float32),
        jax.ShapeDtypeStruct((N_NODES, SD), jnp.float32),
    ],
)


def kernel(nf, ef, edge_index, W_ih_e, W_hh_e, b_ih_e, b_hh_e,
           W_ih_n, W_hh_n, b_ih_n, b_hh_n, W_attn, b_attn):
    ws = jnp.concatenate(
        [W_ih_e[:, :NODE_DIM], W_attn[:, :NODE_DIM],
         jnp.zeros((SD - 49, NODE_DIM), jnp.float32)], axis=0)
    wd = jnp.concatenate(
        [W_ih_e[:, NODE_DIM:], W_attn[:, NODE_DIM:2 * NODE_DIM],
         jnp.zeros((SD - 49, NODE_DIM), jnp.float32)], axis=0)
    bs = jnp.concatenate(
        [b_ih_e, b_attn, jnp.zeros((SD - 49,), jnp.float32)])[None, :]
    ae = W_attn[:, 2 * NODE_DIM:]
    bhh_e = b_hh_e[None, :]
    bih_n = b_ih_n[None, :]
    bhh_n = b_hh_n[None, :]

    src3d = edge_index[0].reshape(NW, NJ, CH)
    dst3d = edge_index[1].reshape(NW, NJ, CH)
    zeros_acc = jnp.zeros((N_PAD, MW), jnp.float32)

    sc_gather, sc_scatter = _build_sc_kernels()

    s_t, d_t = _proj_pass(nf, ws, wd, bs)
    for _ in range(N_ITERS):
        g = sc_gather(s_t, d_t, src3d, dst3d)
        uef, msg = _edge_pass(g, ef, W_hh_e, ae, bhh_e)
        acc = sc_scatter(msg, dst3d, zeros_acc)
        nf, s_t, d_t = _node_pass(acc[0, :N_NODES], acc[1, :N_NODES], nf,
                                  W_ih_n, W_hh_n, bih_n, bhh_n, ws, wd, bs)
        ef = uef
    return (nf, ef)
